# baseline (speedup 1.0000x reference)
import jax
import jax.numpy as jnp
from jax.experimental import pallas as pl
from jax.experimental.pallas import tpu as pltpu

_NCLS = 100000
_B = 1024
_RB = 8
_CHUNK = 4096
_NCH = (_NCLS + _CHUNK - 1) // _CHUNK
_LN2 = 0.6931471805599453


def _row_body(x_ref, tcol_ref, lse_ref, tv_ref):
    tcol = tcol_ref[...]
    m = jnp.full((_RB, 1), -jnp.inf, jnp.float32)
    s = jnp.zeros((_RB, 1), jnp.float32)
    tv = jnp.zeros((_RB, 1), jnp.float32)
    for k in range(_NCH):
        lo = k * _CHUNK
        hi = min(lo + _CHUNK, _NCLS)
        x = x_ref[:, lo:hi]
        bm = jnp.max(x, axis=1, keepdims=True)
        m_new = jnp.maximum(m, bm)
        s = s * jnp.exp(m - m_new) + jnp.sum(
            jnp.exp(x - m_new), axis=1, keepdims=True
        )
        m = m_new
        col_ids = lo + jax.lax.broadcasted_iota(jnp.int32, (1, hi - lo), 1)
        tv = tv + jnp.sum(
            jnp.where(col_ids == tcol, x, 0.0), axis=1, keepdims=True
        )
    lse_ref[...] = m + jnp.log(s)
    tv_ref[...] = tv


def _combine_body(lse_ref, tv_ref, tcol_ref, trow_ref, out_ref):
    nll = lse_ref[...] - tv_ref[...]
    cnt = jnp.sum(
        (tcol_ref[...] == trow_ref[...]).astype(jnp.float32), axis=1, keepdims=True
    )
    u = 1.0 / (1.0 - jnp.exp(cnt * (-_LN2)))
    num = jnp.sum(u * nll, axis=(0, 1), keepdims=True)
    den = jnp.sum(u, axis=(0, 1), keepdims=True)
    out_ref[...] = num / den


def kernel(inputs, targets):
    x = inputs.reshape(_B, _NCLS)
    t = targets.reshape(-1).astype(jnp.int32)
    tcol = t.reshape(_B, 1)
    trow = t.reshape(1, _B)

    lse, tv = pl.pallas_call(
        _row_body,
        grid=(_B // _RB,),
        in_specs=[
            pl.BlockSpec((_RB, _NCLS), lambda i: (i, 0)),
            pl.BlockSpec((_RB, 1), lambda i: (i, 0)),
        ],
        out_specs=[
            pl.BlockSpec((_RB, 1), lambda i: (i, 0)),
            pl.BlockSpec((_RB, 1), lambda i: (i, 0)),
        ],
        out_shape=[
            jax.ShapeDtypeStruct((_B, 1), jnp.float32),
            jax.ShapeDtypeStruct((_B, 1), jnp.float32),
        ],
        compiler_params=pltpu.CompilerParams(
            dimension_semantics=("arbitrary",),
        ),
    )(x, tcol)

    out = pl.pallas_call(
        _combine_body,
        out_shape=jax.ShapeDtypeStruct((1, 1), jnp.float32),
    )(lse, tv, tcol, trow)
    return out[0, 0]
